# baseline (device time: 264481 ns/iter reference)
import jax
import jax.numpy as jnp
from jax import lax
from jax.experimental import pallas as pl
from jax.experimental.pallas import tpu as pltpu

N_DEV = 32
R_MINI = N_DEV
L_MINI = N_DEV - 2


def _cycle_logical_ids():
    devs = [d for d in jax.devices()
            if getattr(d, "core_on_chip", 1) == 1]
    coords = sorted(tuple(d.coords) for d in devs)
    if len(coords) != N_DEV:
        return list(range(N_DEV))
    x0 = min(c[0] for c in coords)
    y0 = min(c[1] for c in coords)
    z0 = min(c[2] for c in coords)
    norm = [(c[0] - x0, c[1] - y0, c[2] - z0) for c in coords]
    if set(norm) != {(x, y, z) for x in range(2) for y in range(4)
                     for z in range(4)}:
        return list(range(N_DEV))
    plane = [(0, 0), (1, 0), (1, 1), (0, 1), (0, 2), (1, 2), (1, 3), (0, 3)]
    logical_of = {}
    lid = 0
    for z in range(4):
        for (x, y) in plane:
            logical_of[(x, y, z)] = lid
            lid += 1
    path = []
    for z in range(4):
        ys = range(4) if z % 2 == 0 else range(3, -1, -1)
        path.extend((y, z) for y in ys)
    cycle = [(0, y, z) for (y, z) in path] + \
            [(1, y, z) for (y, z) in reversed(path)]
    return [logical_of[c] for c in cycle]


def kernel(x, w_mat):
    x = x.astype(jnp.bfloat16)
    w_mat = w_mat.astype(jnp.bfloat16)
    m_per, k = x.shape
    _, n_local = w_mat.shape
    m_total = N_DEV * m_per

    cycle = jnp.asarray(_cycle_logical_ids(), dtype=jnp.int32)
    pos = jnp.zeros((N_DEV,), jnp.int32).at[cycle].set(
        jnp.arange(N_DEV, dtype=jnp.int32))
    my_cyc = jnp.take(pos, lax.axis_index("i"))
    origin_ids = jnp.take(
        cycle, jnp.mod(my_cyc - jnp.arange(N_DEV, dtype=jnp.int32), N_DEV))

    half = m_per // 2
    right_sched = [(s, h) for s in range(N_DEV // 2) for h in (0, 1)]
    left_sched = [(0, 0), (0, 1)] + [
        (s, h) for s in range(N_DEV - 1, N_DEV // 2 + 1, -1) for h in (0, 1)]
    assert len(right_sched) == R_MINI and len(left_sched) == L_MINI

    def body(ids_ref, x_ref, w_ref, out_ref, comm_ref,
             r_send, r_recv, l_send, l_recv,
             hc_buf, hc_in, hc_ss, hc_rs):
        my = lax.axis_index("i")
        left = ids_ref[1]
        right = ids_ref[N_DEV - 1]

        barrier = pltpu.get_barrier_semaphore()
        for nbr in (left, right):
            pl.semaphore_signal(barrier, inc=1, device_id=(nbr,),
                                device_id_type=pl.DeviceIdType.MESH)
        pl.semaphore_wait(barrier, 2)

        comm_ref[0] = x_ref[...]

        def gemm(slot):
            origin = ids_ref[slot]
            out_ref[pl.ds(origin * m_per, m_per), :] = jnp.dot(
                comm_ref[slot], w_ref[...], preferred_element_type=jnp.float32)

        def piece(slot, h):
            return comm_ref.at[slot, pl.ds(h * half, half)]

        rdmas = []
        for j in range(1, R_MINI + 1):
            s, h = right_sched[j - 1]
            rd = pltpu.make_async_remote_copy(
                src_ref=piece(s, h), dst_ref=piece(s + 1, h),
                send_sem=r_send.at[j - 1], recv_sem=r_recv.at[j - 1],
                device_id=(right,), device_id_type=pl.DeviceIdType.MESH)
            rd.start()
            rdmas.append(rd)
            ld = None
            if j <= L_MINI:
                s2, h2 = left_sched[j - 1]
                ld = pltpu.make_async_remote_copy(
                    src_ref=piece(s2, h2),
                    dst_ref=piece((s2 - 1) % N_DEV, h2),
                    send_sem=l_send.at[j - 1], recv_sem=l_recv.at[j - 1],
                    device_id=(left,), device_id_type=pl.DeviceIdType.MESH)
                ld.start()
                rdmas.append(ld)
            if j == 1:
                gemm(0)
            elif j % 2 == 1:
                gemm((j - 1) // 2)
                gemm(N_DEV - (j - 1) // 2)
            rd.wait_recv()
            if ld is not None:
                ld.wait_recv()
        gemm(N_DEV // 2)
        for rd in rdmas:
            rd.wait_send()

        cur = jnp.max(jnp.abs(out_ref[...]))
        hc_buf[...] = cur[None, None]
        sends = []
        for j in range(N_DEV):
            rd = pltpu.make_async_remote_copy(
                src_ref=hc_buf, dst_ref=hc_in.at[my],
                send_sem=hc_ss.at[j], recv_sem=hc_rs.at[my],
                device_id=(j,), device_id_type=pl.DeviceIdType.MESH)
            rd.start()
            sends.append(rd)
        for j in range(N_DEV):
            rcv = pltpu.make_async_remote_copy(
                src_ref=hc_buf, dst_ref=hc_in.at[j],
                send_sem=hc_ss.at[j], recv_sem=hc_rs.at[j],
                device_id=(j,), device_id_type=pl.DeviceIdType.MESH)
            rcv.wait_recv()
        cur = jnp.max(hc_in[...])
        for rd in sends:
            rd.wait_send()

        scale = cur / 448.0
        y = out_ref[...]
        q = (y / scale).astype(jnp.float8_e4m3fn)
        out_ref[...] = q.astype(jnp.float32) * scale

    return pl.pallas_call(
        body,
        out_shape=jax.ShapeDtypeStruct((m_total, n_local), jnp.float32),
        in_specs=[
            pl.BlockSpec(memory_space=pltpu.SMEM),
            pl.BlockSpec(memory_space=pltpu.VMEM),
            pl.BlockSpec(memory_space=pltpu.VMEM),
        ],
        out_specs=pl.BlockSpec(memory_space=pltpu.VMEM),
        scratch_shapes=[
            pltpu.VMEM((N_DEV, m_per, k), jnp.bfloat16),
            pltpu.SemaphoreType.DMA((R_MINI,)),
            pltpu.SemaphoreType.DMA((R_MINI,)),
            pltpu.SemaphoreType.DMA((L_MINI,)),
            pltpu.SemaphoreType.DMA((L_MINI,)),
            pltpu.VMEM((1, 1), jnp.float32),
            pltpu.VMEM((N_DEV, 1, 1), jnp.float32),
            pltpu.SemaphoreType.DMA((N_DEV,)),
            pltpu.SemaphoreType.DMA((N_DEV,)),
        ],
        compiler_params=pltpu.CompilerParams(
            collective_id=0, vmem_limit_bytes=60 * 1024 * 1024),
    )(origin_ids, x, w_mat)


# device time: 212616 ns/iter; 1.2439x vs baseline; 1.2439x over previous
import jax
import jax.numpy as jnp
from jax import lax
from jax.experimental import pallas as pl
from jax.experimental.pallas import tpu as pltpu

N_DEV = 32
R_MINI = N_DEV
L_MINI = N_DEV - 2


def _cycle_logical_ids():
    devs = [d for d in jax.devices()
            if getattr(d, "core_on_chip", 1) == 1]
    coords = sorted(tuple(d.coords) for d in devs)
    if len(coords) != N_DEV:
        return list(range(N_DEV))
    x0 = min(c[0] for c in coords)
    y0 = min(c[1] for c in coords)
    z0 = min(c[2] for c in coords)
    norm = [(c[0] - x0, c[1] - y0, c[2] - z0) for c in coords]
    if set(norm) != {(x, y, z) for x in range(2) for y in range(4)
                     for z in range(4)}:
        return list(range(N_DEV))
    plane = [(0, 0), (1, 0), (1, 1), (0, 1), (0, 2), (1, 2), (1, 3), (0, 3)]
    logical_of = {}
    lid = 0
    for z in range(4):
        for (x, y) in plane:
            logical_of[(x, y, z)] = lid
            lid += 1
    path = []
    for z in range(4):
        ys = range(4) if z % 2 == 0 else range(3, -1, -1)
        path.extend((y, z) for y in ys)
    cycle = [(0, y, z) for (y, z) in path] + \
            [(1, y, z) for (y, z) in reversed(path)]
    return [logical_of[c] for c in cycle]


def kernel(x, w_mat):
    x = x.astype(jnp.bfloat16)
    w_mat = w_mat.astype(jnp.bfloat16)
    m_per, k = x.shape
    _, n_local = w_mat.shape
    m_total = N_DEV * m_per

    cycle = jnp.asarray(_cycle_logical_ids(), dtype=jnp.int32)
    pos = jnp.zeros((N_DEV,), jnp.int32).at[cycle].set(
        jnp.arange(N_DEV, dtype=jnp.int32))
    my_cyc = jnp.take(pos, lax.axis_index("i"))
    origin_ids = jnp.take(
        cycle, jnp.mod(my_cyc - jnp.arange(N_DEV, dtype=jnp.int32), N_DEV))

    half = m_per // 2
    right_sched = [(s, h) for s in range(N_DEV // 2) for h in (0, 1)]
    left_sched = [(0, 0), (0, 1)] + [
        (s, h) for s in range(N_DEV - 1, N_DEV // 2 + 1, -1) for h in (0, 1)]
    assert len(right_sched) == R_MINI and len(left_sched) == L_MINI

    def body(ids_ref, x_ref, w_ref, out_ref, comm_ref,
             r_send, r_recv, l_send, l_recv,
             hc_buf, hc_in, hc_ss, hc_rs):
        my = lax.axis_index("i")
        left = ids_ref[1]
        right = ids_ref[N_DEV - 1]

        barrier = pltpu.get_barrier_semaphore()
        for nbr in (left, right):
            pl.semaphore_signal(barrier, inc=1, device_id=(nbr,),
                                device_id_type=pl.DeviceIdType.MESH)
        pl.semaphore_wait(barrier, 2)

        comm_ref[0] = x_ref[...]

        amax_cell = [jnp.float32(0.0)]

        def gemm(slot):
            origin = ids_ref[slot]
            res = jnp.dot(comm_ref[slot], w_ref[...],
                          preferred_element_type=jnp.float32)
            out_ref[pl.ds(origin * m_per, m_per), :] = res
            amax_cell[0] = jnp.maximum(amax_cell[0], jnp.max(jnp.abs(res)))

        def piece(slot, h):
            return comm_ref.at[slot, pl.ds(h * half, half)]

        rdmas = []
        pend_r = pend_l = None
        for j in range(1, R_MINI + 1):
            s, h = right_sched[j - 1]
            rd = pltpu.make_async_remote_copy(
                src_ref=piece(s, h), dst_ref=piece(s + 1, h),
                send_sem=r_send.at[j - 1], recv_sem=r_recv.at[j - 1],
                device_id=(right,), device_id_type=pl.DeviceIdType.MESH)
            rd.start()
            rdmas.append(rd)
            ld = None
            if j <= L_MINI:
                s2, h2 = left_sched[j - 1]
                ld = pltpu.make_async_remote_copy(
                    src_ref=piece(s2, h2),
                    dst_ref=piece((s2 - 1) % N_DEV, h2),
                    send_sem=l_send.at[j - 1], recv_sem=l_recv.at[j - 1],
                    device_id=(left,), device_id_type=pl.DeviceIdType.MESH)
                ld.start()
                rdmas.append(ld)
            if j == 1:
                gemm(0)
            elif j % 2 == 0 and j >= 4:
                gemm((j - 2) // 2)
                gemm(N_DEV - (j - 2) // 2)
            if pend_r is not None:
                pend_r.wait_recv()
            if pend_l is not None:
                pend_l.wait_recv()
            pend_r, pend_l = rd, ld
        pend_r.wait_recv()
        if pend_l is not None:
            pend_l.wait_recv()
        gemm(N_DEV // 2)
        for rd in rdmas:
            rd.wait_send()

        cur = amax_cell[0]
        hc_buf[...] = cur[None, None]
        sends = []
        for j in range(N_DEV):
            rd = pltpu.make_async_remote_copy(
                src_ref=hc_buf, dst_ref=hc_in.at[my],
                send_sem=hc_ss.at[j], recv_sem=hc_rs.at[my],
                device_id=(j,), device_id_type=pl.DeviceIdType.MESH)
            rd.start()
            sends.append(rd)
        for j in range(N_DEV):
            rcv = pltpu.make_async_remote_copy(
                src_ref=hc_buf, dst_ref=hc_in.at[j],
                send_sem=hc_ss.at[j], recv_sem=hc_rs.at[j],
                device_id=(j,), device_id_type=pl.DeviceIdType.MESH)
            rcv.wait_recv()
        cur = jnp.max(hc_in[...])
        for rd in sends:
            rd.wait_send()

        scale = cur / 448.0
        y = out_ref[...]
        q = (y / scale).astype(jnp.float8_e4m3fn)
        out_ref[...] = q.astype(jnp.float32) * scale

    return pl.pallas_call(
        body,
        out_shape=jax.ShapeDtypeStruct((m_total, n_local), jnp.float32),
        in_specs=[
            pl.BlockSpec(memory_space=pltpu.SMEM),
            pl.BlockSpec(memory_space=pltpu.VMEM),
            pl.BlockSpec(memory_space=pltpu.VMEM),
        ],
        out_specs=pl.BlockSpec(memory_space=pltpu.VMEM),
        scratch_shapes=[
            pltpu.VMEM((N_DEV, m_per, k), jnp.bfloat16),
            pltpu.SemaphoreType.DMA((R_MINI,)),
            pltpu.SemaphoreType.DMA((R_MINI,)),
            pltpu.SemaphoreType.DMA((L_MINI,)),
            pltpu.SemaphoreType.DMA((L_MINI,)),
            pltpu.VMEM((1, 1), jnp.float32),
            pltpu.VMEM((N_DEV, 1, 1), jnp.float32),
            pltpu.SemaphoreType.DMA((N_DEV,)),
            pltpu.SemaphoreType.DMA((N_DEV,)),
        ],
        compiler_params=pltpu.CompilerParams(
            collective_id=0, vmem_limit_bytes=60 * 1024 * 1024),
    )(origin_ids, x, w_mat)


# device time: 208406 ns/iter; 1.2691x vs baseline; 1.0202x over previous
import jax
import jax.numpy as jnp
from jax import lax
from jax.experimental import pallas as pl
from jax.experimental.pallas import tpu as pltpu

N_DEV = 32
N_MINI = N_DEV - 1


def _cycle_logical_ids():
    devs = [d for d in jax.devices()
            if getattr(d, "core_on_chip", 1) == 1]
    coords = sorted(tuple(d.coords) for d in devs)
    if len(coords) != N_DEV:
        return list(range(N_DEV))
    x0 = min(c[0] for c in coords)
    y0 = min(c[1] for c in coords)
    z0 = min(c[2] for c in coords)
    norm = [(c[0] - x0, c[1] - y0, c[2] - z0) for c in coords]
    if set(norm) != {(x, y, z) for x in range(2) for y in range(4)
                     for z in range(4)}:
        return list(range(N_DEV))
    plane = [(0, 0), (1, 0), (1, 1), (0, 1), (0, 2), (1, 2), (1, 3), (0, 3)]
    logical_of = {}
    lid = 0
    for z in range(4):
        for (x, y) in plane:
            logical_of[(x, y, z)] = lid
            lid += 1
    path = []
    for z in range(4):
        ys = range(4) if z % 2 == 0 else range(3, -1, -1)
        path.extend((y, z) for y in ys)
    cycle = [(0, y, z) for (y, z) in path] + \
            [(1, y, z) for (y, z) in reversed(path)]
    return [logical_of[c] for c in cycle]


def kernel(x, w_mat):
    x = x.astype(jnp.bfloat16)
    w_mat = w_mat.astype(jnp.bfloat16)
    m_per, k = x.shape
    _, n_local = w_mat.shape
    m_total = N_DEV * m_per

    cycle = jnp.asarray(_cycle_logical_ids(), dtype=jnp.int32)
    pos = jnp.zeros((N_DEV,), jnp.int32).at[cycle].set(
        jnp.arange(N_DEV, dtype=jnp.int32))
    my_cyc = jnp.take(pos, lax.axis_index("i"))
    origin_ids = jnp.take(
        cycle, jnp.mod(my_cyc - jnp.arange(N_DEV, dtype=jnp.int32), N_DEV))

    half = m_per // 2
    right_sched = [(s, h) for s in range(N_DEV // 2 - 1) for h in (0, 1)] \
        + [(N_DEV // 2 - 1, 0)]
    left_sched = [(0, 0), (0, 1)] + [
        (s, h) for s in range(N_DEV - 1, N_DEV // 2 + 1, -1) for h in (0, 1)
    ] + [(N_DEV // 2 + 1, 1)]
    assert len(right_sched) == N_MINI and len(left_sched) == N_MINI

    def body(ids_ref, x_ref, w_ref, out_ref, comm_ref,
             r_send, r_recv, l_send, l_recv,
             hc_buf, hc_in, hc_ss, hc_rs):
        my = lax.axis_index("i")
        left = ids_ref[1]
        right = ids_ref[N_DEV - 1]

        barrier = pltpu.get_barrier_semaphore()
        for nbr in (left, right):
            pl.semaphore_signal(barrier, inc=1, device_id=(nbr,),
                                device_id_type=pl.DeviceIdType.MESH)
        pl.semaphore_wait(barrier, 2)

        comm_ref[0] = x_ref[...]

        amax_cell = [jnp.float32(0.0)]

        def gemm(slot):
            origin = ids_ref[slot]
            res = jnp.dot(comm_ref[slot], w_ref[...],
                          preferred_element_type=jnp.float32)
            out_ref[pl.ds(origin * m_per, m_per), :] = res
            amax_cell[0] = jnp.maximum(amax_cell[0], jnp.max(jnp.abs(res)))

        def piece(slot, h):
            return comm_ref.at[slot, pl.ds(h * half, half)]

        def send_right(j):
            s, h = right_sched[j - 1]
            rd = pltpu.make_async_remote_copy(
                src_ref=piece(s, h), dst_ref=piece(s + 1, h),
                send_sem=r_send.at[j - 1], recv_sem=r_recv.at[j - 1],
                device_id=(right,), device_id_type=pl.DeviceIdType.MESH)
            rd.start()
            return rd

        def send_left(j):
            s, h = left_sched[j - 1]
            ld = pltpu.make_async_remote_copy(
                src_ref=piece(s, h), dst_ref=piece((s - 1) % N_DEV, h),
                send_sem=l_send.at[j - 1], recv_sem=l_recv.at[j - 1],
                device_id=(left,), device_id_type=pl.DeviceIdType.MESH)
            ld.start()
            return ld

        rdmas = []
        pend_r = pend_l = None
        for j in range(1, N_MINI):
            rd = send_right(j)
            ld = send_left(j)
            rdmas += [rd, ld]
            if j == 1:
                gemm(0)
            elif j % 2 == 0 and j >= 4:
                gemm((j - 2) // 2)
                gemm(N_DEV - (j - 2) // 2)
            if pend_r is not None:
                pend_r.wait_recv()
            if pend_l is not None:
                pend_l.wait_recv()
            pend_r, pend_l = rd, ld
        pend_l.wait_recv()
        ld = send_left(N_MINI)
        rd = send_right(N_MINI)
        rdmas += [rd, ld]
        pend_r.wait_recv()
        gemm(N_DEV // 2 - 1)
        gemm(N_DEV // 2 + 1)
        rd.wait_recv()
        ld.wait_recv()

        s16 = N_DEV // 2
        res16 = jnp.dot(comm_ref[s16], w_ref[...],
                        preferred_element_type=jnp.float32)
        cur = jnp.maximum(amax_cell[0], jnp.max(jnp.abs(res16)))
        hc_buf[...] = cur[None, None]

        sends = []
        for j in range(N_DEV):
            srd = pltpu.make_async_remote_copy(
                src_ref=hc_buf, dst_ref=hc_in.at[my],
                send_sem=hc_ss.at[j], recv_sem=hc_rs.at[my],
                device_id=(j,), device_id_type=pl.DeviceIdType.MESH)
            srd.start()
            sends.append(srd)
        out_ref[pl.ds(ids_ref[s16] * m_per, m_per), :] = res16
        for j in range(N_DEV):
            rcv = pltpu.make_async_remote_copy(
                src_ref=hc_buf, dst_ref=hc_in.at[j],
                send_sem=hc_ss.at[j], recv_sem=hc_rs.at[j],
                device_id=(j,), device_id_type=pl.DeviceIdType.MESH)
            rcv.wait_recv()
        cur = jnp.max(hc_in[...])
        for srd in sends:
            srd.wait_send()
        for rd in rdmas:
            rd.wait_send()

        scale = cur / 448.0
        y = out_ref[...]
        q = (y / scale).astype(jnp.float8_e4m3fn)
        out_ref[...] = q.astype(jnp.float32) * scale

    return pl.pallas_call(
        body,
        out_shape=jax.ShapeDtypeStruct((m_total, n_local), jnp.float32),
        in_specs=[
            pl.BlockSpec(memory_space=pltpu.SMEM),
            pl.BlockSpec(memory_space=pltpu.VMEM),
            pl.BlockSpec(memory_space=pltpu.VMEM),
        ],
        out_specs=pl.BlockSpec(memory_space=pltpu.VMEM),
        scratch_shapes=[
            pltpu.VMEM((N_DEV, m_per, k), jnp.bfloat16),
            pltpu.SemaphoreType.DMA((N_MINI,)),
            pltpu.SemaphoreType.DMA((N_MINI,)),
            pltpu.SemaphoreType.DMA((N_MINI,)),
            pltpu.SemaphoreType.DMA((N_MINI,)),
            pltpu.VMEM((1, 1), jnp.float32),
            pltpu.VMEM((N_DEV, 1, 1), jnp.float32),
            pltpu.SemaphoreType.DMA((N_DEV,)),
            pltpu.SemaphoreType.DMA((N_DEV,)),
        ],
        compiler_params=pltpu.CompilerParams(
            collective_id=0, vmem_limit_bytes=60 * 1024 * 1024),
    )(origin_ids, x, w_mat)


# device time: 199988 ns/iter; 1.3225x vs baseline; 1.0421x over previous
import jax
import jax.numpy as jnp
from jax import lax
from jax.experimental import pallas as pl
from jax.experimental.pallas import tpu as pltpu

N_DEV = 32
N_MINI = N_DEV - 1


def _cycle_logical_ids():
    devs = [d for d in jax.devices()
            if getattr(d, "core_on_chip", 1) == 1]
    coords = sorted(tuple(d.coords) for d in devs)
    if len(coords) != N_DEV:
        return list(range(N_DEV))
    x0 = min(c[0] for c in coords)
    y0 = min(c[1] for c in coords)
    z0 = min(c[2] for c in coords)
    norm = [(c[0] - x0, c[1] - y0, c[2] - z0) for c in coords]
    if set(norm) != {(x, y, z) for x in range(2) for y in range(4)
                     for z in range(4)}:
        return list(range(N_DEV))
    plane = [(0, 0), (1, 0), (1, 1), (0, 1), (0, 2), (1, 2), (1, 3), (0, 3)]
    logical_of = {}
    lid = 0
    for z in range(4):
        for (x, y) in plane:
            logical_of[(x, y, z)] = lid
            lid += 1
    path = []
    for z in range(4):
        ys = range(4) if z % 2 == 0 else range(3, -1, -1)
        path.extend((y, z) for y in ys)
    cycle = [(0, y, z) for (y, z) in path] + \
            [(1, y, z) for (y, z) in reversed(path)]
    return [logical_of[c] for c in cycle]


def kernel(x, w_mat):
    m_per, k = x.shape
    _, n_local = w_mat.shape
    m_total = N_DEV * m_per

    cycle_list = _cycle_logical_ids()
    pos_list = [0] * N_DEV
    for p, lid in enumerate(cycle_list):
        pos_list[lid] = p
    cycle_arr = jnp.asarray(cycle_list, dtype=jnp.int32)
    pos_arr = jnp.asarray(pos_list, dtype=jnp.int32)

    half = m_per // 2
    right_sched = [(s, h) for s in range(N_DEV // 2 - 1) for h in (0, 1)] \
        + [(N_DEV // 2 - 1, 0)]
    left_sched = [(0, 0), (0, 1)] + [
        (s, h) for s in range(N_DEV - 1, N_DEV // 2 + 1, -1) for h in (0, 1)
    ] + [(N_DEV // 2 + 1, 1)]
    assert len(right_sched) == N_MINI and len(left_sched) == N_MINI

    def body(cycle_ref, pos_ref, x_ref, w_ref, out_ref, comm_ref, w_bf,
             r_send, r_recv, l_send, l_recv,
             hc_buf, hc_in, hc_ss, hc_rs):
        my = lax.axis_index("i")
        my_cyc = pos_ref[my]

        def oid(d):
            return cycle_ref[lax.rem(my_cyc - d + N_DEV, N_DEV)]

        left = oid(1)
        right = oid(N_DEV - 1)

        barrier = pltpu.get_barrier_semaphore()
        for nbr in (left, right):
            pl.semaphore_signal(barrier, inc=1, device_id=(nbr,),
                                device_id_type=pl.DeviceIdType.MESH)
        comm_ref[0] = x_ref[...].astype(jnp.bfloat16)
        w_bf[...] = w_ref[...].astype(jnp.bfloat16)
        pl.semaphore_wait(barrier, 2)

        amax_cell = [jnp.float32(0.0)]

        def gemm(slot):
            origin = oid(slot)
            res = jnp.dot(comm_ref[slot], w_bf[...],
                          preferred_element_type=jnp.float32)
            out_ref[pl.ds(origin * m_per, m_per), :] = res
            amax_cell[0] = jnp.maximum(amax_cell[0], jnp.max(jnp.abs(res)))

        def piece(slot, h):
            return comm_ref.at[slot, pl.ds(h * half, half)]

        def send_right(j):
            s, h = right_sched[j - 1]
            rd = pltpu.make_async_remote_copy(
                src_ref=piece(s, h), dst_ref=piece(s + 1, h),
                send_sem=r_send.at[j - 1], recv_sem=r_recv.at[j - 1],
                device_id=(right,), device_id_type=pl.DeviceIdType.MESH)
            rd.start()
            return rd

        def send_left(j):
            s, h = left_sched[j - 1]
            ld = pltpu.make_async_remote_copy(
                src_ref=piece(s, h), dst_ref=piece((s - 1) % N_DEV, h),
                send_sem=l_send.at[j - 1], recv_sem=l_recv.at[j - 1],
                device_id=(left,), device_id_type=pl.DeviceIdType.MESH)
            ld.start()
            return ld

        rdmas = []
        pend_r = pend_l = None
        for j in range(1, N_MINI):
            rd = send_right(j)
            ld = send_left(j)
            rdmas += [rd, ld]
            if j == 1:
                gemm(0)
            elif j % 2 == 0 and j >= 4:
                gemm((j - 2) // 2)
                gemm(N_DEV - (j - 2) // 2)
            if pend_r is not None:
                pend_r.wait_recv()
            if pend_l is not None:
                pend_l.wait_recv()
            pend_r, pend_l = rd, ld
        pend_l.wait_recv()
        ld = send_left(N_MINI)
        rd = send_right(N_MINI)
        rdmas += [rd, ld]
        pend_r.wait_recv()
        gemm(N_DEV // 2 - 1)
        gemm(N_DEV // 2 + 1)
        rd.wait_recv()
        ld.wait_recv()

        s16 = N_DEV // 2
        res16 = jnp.dot(comm_ref[s16], w_bf[...],
                        preferred_element_type=jnp.float32)
        cur = jnp.maximum(amax_cell[0], jnp.max(jnp.abs(res16)))
        hc_buf[...] = cur[None, None]

        sends = []
        for j in range(N_DEV):
            srd = pltpu.make_async_remote_copy(
                src_ref=hc_buf, dst_ref=hc_in.at[my],
                send_sem=hc_ss.at[j], recv_sem=hc_rs.at[my],
                device_id=(j,), device_id_type=pl.DeviceIdType.MESH)
            srd.start()
            sends.append(srd)
        out_ref[pl.ds(oid(s16) * m_per, m_per), :] = res16
        for j in range(N_DEV):
            rcv = pltpu.make_async_remote_copy(
                src_ref=hc_buf, dst_ref=hc_in.at[j],
                send_sem=hc_ss.at[j], recv_sem=hc_rs.at[j],
                device_id=(j,), device_id_type=pl.DeviceIdType.MESH)
            rcv.wait_recv()
        cur = jnp.max(hc_in[...])
        for srd in sends:
            srd.wait_send()
        for rd in rdmas:
            rd.wait_send()

        scale = cur / 448.0
        y = out_ref[...]
        q = (y / scale).astype(jnp.float8_e4m3fn)
        out_ref[...] = q.astype(jnp.float32) * scale

    return pl.pallas_call(
        body,
        out_shape=jax.ShapeDtypeStruct((m_total, n_local), jnp.float32),
        in_specs=[
            pl.BlockSpec(memory_space=pltpu.SMEM),
            pl.BlockSpec(memory_space=pltpu.SMEM),
            pl.BlockSpec(memory_space=pltpu.VMEM),
            pl.BlockSpec(memory_space=pltpu.VMEM),
        ],
        out_specs=pl.BlockSpec(memory_space=pltpu.VMEM),
        scratch_shapes=[
            pltpu.VMEM((N_DEV, m_per, k), jnp.bfloat16),
            pltpu.VMEM((k, n_local), jnp.bfloat16),
            pltpu.SemaphoreType.DMA((N_MINI,)),
            pltpu.SemaphoreType.DMA((N_MINI,)),
            pltpu.SemaphoreType.DMA((N_MINI,)),
            pltpu.SemaphoreType.DMA((N_MINI,)),
            pltpu.VMEM((1, 1), jnp.float32),
            pltpu.VMEM((N_DEV, 1, 1), jnp.float32),
            pltpu.SemaphoreType.DMA((N_DEV,)),
            pltpu.SemaphoreType.DMA((N_DEV,)),
        ],
        compiler_params=pltpu.CompilerParams(
            collective_id=0, vmem_limit_bytes=60 * 1024 * 1024),
    )(cycle_arr, pos_arr, x, w_mat)
